# baseline (speedup 1.0000x reference)
import functools

import jax
import jax.numpy as jnp
from jax import lax
from jax.experimental import pallas as pl
from jax.experimental.pallas import tpu as pltpu
from jax.experimental.pallas import tpu_sc as plsc

INPUT_DIM = 4096
PATCH_NUM = 16
KSZ = INPUT_DIM // PATCH_NUM
OUT_CHANNEL = 192
C2 = 2 * OUT_CHANNEL
EXPERT_NUM = 8
B = 2048

TB = 128
NB = B // TB

GB = 128
SORT_CAP = B + EXPERT_NUM * GB
NBLK = SORT_CAP // GB
BE_PAD = 48

NW = 32
TPW = B // NW
NCH = TPW // 16


def _router_body(x_ref, wr_ref, sel0_ref, gate_ref, idx_ref, lb_ref, acc_ref):
    i = pl.program_id(0)
    xb = x_ref[...].reshape(TB, INPUT_DIM)
    ps = xb[:, 0:KSZ]
    for p_i in range(1, PATCH_NUM):
        ps = ps + xb[:, p_i * KSZ:(p_i + 1) * KSZ]
    select = jnp.dot(ps.astype(jnp.bfloat16),
                     wr_ref[...].astype(jnp.bfloat16),
                     preferred_element_type=jnp.float32)

    gate = jnp.max(select, axis=1, keepdims=True)
    iota8 = lax.broadcasted_iota(jnp.int32, (TB, EXPERT_NUM), 1)
    idx = jnp.min(jnp.where(select == gate, iota8, EXPERT_NUM), axis=1,
                  keepdims=True)
    mask = (iota8 == idx).astype(jnp.float32)
    sel0_ref[...] = mask * (gate != 0.0).astype(jnp.float32)
    gate_ref[...] = gate
    idx_ref[...] = idx

    @pl.when(i == 0)
    def _init():
        acc_ref[...] = jnp.zeros_like(acc_ref)

    acc_ref[0:1, :] += jnp.sum(select, axis=0, keepdims=True)
    acc_ref[1:2, :] += jnp.sum(mask, axis=0, keepdims=True)

    @pl.when(i == NB - 1)
    def _final():
        s = acc_ref[0:1, :] * acc_ref[1:2, :]
        lb = jnp.sum(s) * (float(EXPERT_NUM ** 2) /
                           (float(B) * float(B) * float(EXPERT_NUM)))
        lb_ref[...] = jnp.broadcast_to(lb, (1, 1))


def _router_call(x3, wr_full):
    return pl.pallas_call(
        _router_body,
        grid=(NB,),
        in_specs=[
            pl.BlockSpec((TB, 1, INPUT_DIM), lambda i: (i, 0, 0)),
            pl.BlockSpec((KSZ, EXPERT_NUM), lambda i: (0, 0)),
        ],
        out_specs=[
            pl.BlockSpec((TB, EXPERT_NUM), lambda i: (i, 0)),
            pl.BlockSpec((TB, 1), lambda i: (i, 0)),
            pl.BlockSpec((TB, 1), lambda i: (i, 0)),
            pl.BlockSpec((1, 1), lambda i: (0, 0)),
        ],
        out_shape=[
            jax.ShapeDtypeStruct((B, EXPERT_NUM), jnp.float32),
            jax.ShapeDtypeStruct((B, 1), jnp.float32),
            jax.ShapeDtypeStruct((B, 1), jnp.int32),
            jax.ShapeDtypeStruct((1, 1), jnp.float32),
        ],
        scratch_shapes=[pltpu.VMEM((2, EXPERT_NUM), jnp.float32)],
    )(x3, wr_full)


def _lane_iota():
    return lax.iota(jnp.int32, 16)


def _vgather(v, idx):
    dn = lax.GatherDimensionNumbers(offset_dims=(), collapsed_slice_dims=(0,),
                                    start_index_map=(0,))
    return lax.gather(v, idx[:, None], dn, slice_sizes=(1,),
                      mode=lax.GatherScatterMode.PROMISE_IN_BOUNDS)


def _vsum_splat(v):
    lanes = _lane_iota()
    s = v
    for d in (8, 4, 2, 1):
        s = s + _vgather(s, (lanes + d) & 15)
    return s


def _vcumsum(v):
    lanes = _lane_iota()
    s = v
    for d in (1, 2, 4, 8):
        sh = _vgather(s, jnp.maximum(lanes - d, 0))
        s = s + jnp.where(lanes >= d, sh, 0)
    return s


def _splat_lane(v, lane):
    return _vgather(v, jnp.full((16,), lane, jnp.int32))


def _sc_sort_body(idx_hbm, x_hbm, xs_hbm, dest_hbm, be_hbm,
                  idx_v, dest_v, dest2_v, rows_a, rows_b, be_v,
                  gsem_a, gsem_b, ssem_a, ssem_b):
    nc = jax.lax.axis_size("c")
    wid = lax.axis_index("s") * nc + lax.axis_index("c")
    base = wid * TPW

    pltpu.sync_copy(idx_hbm, idx_v)

    lanes = _lane_iota()
    z16 = jnp.zeros((16,), jnp.int32)

    def hist_step(i, carry):
        tv, pv = carry
        v = idx_v[pl.ds(i * 16, 16)]
        before = jnp.where((i * 16) < base, 1, 0)
        tv2, pv2 = [], []
        for e in range(EXPERT_NUM):
            mi = jnp.where(v == e, 1, 0)
            tv2.append(tv[e] + mi)
            pv2.append(pv[e] + mi * before)
        return tuple(tv2), tuple(pv2)

    tv, pv = lax.fori_loop(0, B // 16, hist_step,
                           ((z16,) * EXPERT_NUM, (z16,) * EXPERT_NUM))
    total = z16
    prior = z16
    for e in range(EXPERT_NUM):
        total = jnp.where(lanes == e, _vsum_splat(tv[e]), total)
        prior = jnp.where(lanes == e, _vsum_splat(pv[e]), prior)

    padded = (total + (GB - 1)) & jnp.int32(~(GB - 1))
    cum = _vcumsum(padded)
    base_e = cum - padded
    pos0 = base_e + prior

    carry = z16
    for c in range(NCH):
        v = idx_v[pl.ds(base + c * 16, 16)]
        dest_c = z16
        for e in range(EXPERT_NUM):
            m = v == e
            mi = jnp.where(m, 1, 0)
            excl = _vcumsum(mi) - mi
            start_e = _splat_lane(pos0 + carry, e)
            dest_c = jnp.where(m, start_e + excl, dest_c)
            cnt = _vsum_splat(mi)
            carry = jnp.where(lanes == e, carry + cnt, carry)
        dest_v[pl.ds(c * 16, 16)] = dest_c
        dest2_v[c, :] = dest_c

    pltpu.sync_copy(dest_v, dest_hbm.at[pl.ds(base, TPW)])

    nsub = TPW // 8
    for j in range(nsub):
        buf = rows_a if j % 2 == 0 else rows_b
        gsem = gsem_a if j % 2 == 0 else gsem_b
        ssem = ssem_a if j % 2 == 0 else ssem_b
        idx8 = dest2_v.at[(j - 2) // 2, pl.ds(((j - 2) % 2) * 8, 8)]
        if j >= 2:
            pltpu.make_async_copy(buf, xs_hbm.at[idx8], ssem).wait()
        pltpu.async_copy(x_hbm.at[pl.ds(base + j * 8, 8)], buf, gsem).wait()
        pltpu.async_copy(
            buf, xs_hbm.at[dest2_v.at[j // 2, pl.ds((j % 2) * 8, 8)]], ssem)
    for j in range(nsub - 2, nsub):
        buf = rows_a if j % 2 == 0 else rows_b
        ssem = ssem_a if j % 2 == 0 else ssem_b
        pltpu.make_async_copy(
            buf, xs_hbm.at[dest2_v.at[j // 2, pl.ds((j % 2) * 8, 8)]],
            ssem).wait()

    @pl.when(wid == 0)
    def _be():
        for c3 in range(BE_PAD // 16):
            posv = (lanes + c3 * 16) * GB
            acc = jnp.zeros((16,), jnp.int32)
            for e in range(EXPERT_NUM):
                ce = _splat_lane(cum, e)
                acc = acc + jnp.where(posv >= ce, 1, 0)
            be_v[pl.ds(c3 * 16, 16)] = jnp.minimum(acc, EXPERT_NUM - 1)
        pltpu.sync_copy(be_v, be_hbm)


def _sc_sort_call(idx_flat, x2):
    mesh = plsc.VectorSubcoreMesh(core_axis_name="c", subcore_axis_name="s")
    f = pl.kernel(
        _sc_sort_body,
        compiler_params=pltpu.CompilerParams(use_tc_tiling_on_sc=True),
        out_type=[
            jax.ShapeDtypeStruct((SORT_CAP, 1, INPUT_DIM), jnp.float32),
            jax.ShapeDtypeStruct((B,), jnp.int32),
            jax.ShapeDtypeStruct((BE_PAD,), jnp.int32),
        ],
        mesh=mesh,
        scratch_types=[
            pltpu.VMEM((B,), jnp.int32),
            pltpu.VMEM((TPW,), jnp.int32),
            pltpu.VMEM((NCH, 16), jnp.int32),
            pltpu.VMEM((8, 1, INPUT_DIM), jnp.float32),
            pltpu.VMEM((8, 1, INPUT_DIM), jnp.float32),
            pltpu.VMEM((BE_PAD,), jnp.int32),
            pltpu.SemaphoreType.DMA,
            pltpu.SemaphoreType.DMA,
            pltpu.SemaphoreType.DMA,
            pltpu.SemaphoreType.DMA,
        ],
    )
    return f(idx_flat, x2)


def _grouped_body(be_sref, x4_ref, wt_ref, b_ref, out_ref):
    xb = x4_ref[...].reshape(GB, INPUT_DIM).astype(jnp.bfloat16)
    wt = wt_ref[0].astype(jnp.bfloat16)
    bias = b_ref[0, 0, :][None, :]
    h3s = jnp.zeros((GB, C2), jnp.float32)
    for p_i in range(PATCH_NUM):
        m = jnp.dot(xb[:, p_i * KSZ:(p_i + 1) * KSZ], wt,
                    preferred_element_type=jnp.float32)
        hh = m + bias
        h3s = h3s + hh * hh * hh
    iota_c = lax.broadcasted_iota(jnp.int32, (C2, 2), 0)
    iota_h = lax.broadcasted_iota(jnp.int32, (C2, 2), 1)
    sel_mat = ((iota_c < OUT_CHANNEL) == (iota_h == 0)).astype(jnp.float32)
    out_ref[...] = jnp.dot(h3s, sel_mat, preferred_element_type=jnp.float32)


def _grouped_call(blk_expert, x4, conv_wt, conv_b):
    grid_spec = pltpu.PrefetchScalarGridSpec(
        num_scalar_prefetch=1,
        grid=(NBLK,),
        in_specs=[
            pl.BlockSpec((GB, 1, INPUT_DIM), lambda i, be: (i, 0, 0)),
            pl.BlockSpec((1, KSZ, C2), lambda i, be: (be[i], 0, 0)),
            pl.BlockSpec((1, 1, C2), lambda i, be: (be[i], 0, 0)),
        ],
        out_specs=pl.BlockSpec((GB, 2), lambda i, be: (i, 0)),
    )
    return pl.pallas_call(
        _grouped_body,
        grid_spec=grid_spec,
        out_shape=jax.ShapeDtypeStruct((SORT_CAP, 2), jnp.float32),
    )(blk_expert, x4, conv_wt, conv_b)


def _combine_body(dest_ref, gate_ref, s_ref, out_ref):
    dest = dest_ref[...]
    iota_s = lax.broadcasted_iota(jnp.int32, (TB, SORT_CAP), 1)
    onehot = (iota_s == dest).astype(jnp.bfloat16)
    sp = jnp.dot(onehot, s_ref[...].astype(jnp.bfloat16),
                 preferred_element_type=jnp.float32)
    logits = gate_ref[...] * sp
    mx = jnp.max(logits, axis=1, keepdims=True)
    p = jnp.exp(logits - mx)
    out_ref[...] = p / jnp.sum(p, axis=1, keepdims=True)


def _combine_call(dest, gate, s_sorted):
    return pl.pallas_call(
        _combine_body,
        grid=(NB,),
        in_specs=[
            pl.BlockSpec((TB, 1), lambda i: (i, 0)),
            pl.BlockSpec((TB, 1), lambda i: (i, 0)),
            pl.BlockSpec((SORT_CAP, 2), lambda i: (0, 0)),
        ],
        out_specs=pl.BlockSpec((TB, 2), lambda i: (i, 0)),
        out_shape=jax.ShapeDtypeStruct((B, 2), jnp.float32),
    )(dest, gate, s_sorted)


@jax.jit
def _moe_pipeline(x, wr_full, conv_wt, conv_b):
    sel0, gate, idx, lb = _router_call(x, wr_full)
    x_sorted, dest, blk_expert = _sc_sort_call(idx.reshape(B), x)
    s_sorted = _grouped_call(blk_expert, x_sorted, conv_wt,
                             conv_b.reshape(EXPERT_NUM, 1, C2))
    out = _combine_call(dest.reshape(B, 1), gate, s_sorted)
    return out, sel0, lb


def kernel(x, router_w, conv_w, conv_b):
    wr_full = router_w[:, 0, :].T
    conv_wt = conv_w[:, :, 0, :].transpose(0, 2, 1)
    out, sel0, lb = _moe_pipeline(x, wr_full, conv_wt, conv_b)
    return out, sel0, lb.reshape(())


# speedup vs baseline: 1.1128x; 1.1128x over previous
import functools

import jax
import jax.numpy as jnp
from jax import lax
from jax.experimental import pallas as pl
from jax.experimental.pallas import tpu as pltpu
from jax.experimental.pallas import tpu_sc as plsc

INPUT_DIM = 4096
PATCH_NUM = 16
KSZ = INPUT_DIM // PATCH_NUM
OUT_CHANNEL = 192
C2 = 2 * OUT_CHANNEL
EXPERT_NUM = 8
B = 2048

TB = 256
NB = B // TB

GB = 64
SORT_CAP = B + EXPERT_NUM * GB
NBLK = SORT_CAP // GB
BE_PAD = 48

NW = 32
TPW = B // NW
NCH = TPW // 16


def _router_body(x_ref, wr_ref, sel0_ref, gate_ref, idx_ref, lb_ref, acc_ref):
    i = pl.program_id(0)
    xb = x_ref[...].reshape(TB, INPUT_DIM)
    ps = xb[:, 0:KSZ]
    for p_i in range(1, PATCH_NUM):
        ps = ps + xb[:, p_i * KSZ:(p_i + 1) * KSZ]
    select = jnp.dot(ps.astype(jnp.bfloat16),
                     wr_ref[...].astype(jnp.bfloat16),
                     preferred_element_type=jnp.float32)

    gate = jnp.max(select, axis=1, keepdims=True)
    iota8 = lax.broadcasted_iota(jnp.int32, (TB, EXPERT_NUM), 1)
    idx = jnp.min(jnp.where(select == gate, iota8, EXPERT_NUM), axis=1,
                  keepdims=True)
    mask = (iota8 == idx).astype(jnp.float32)
    sel0_ref[...] = mask * (gate != 0.0).astype(jnp.float32)
    gate_ref[...] = gate
    idx_ref[...] = idx

    @pl.when(i == 0)
    def _init():
        acc_ref[...] = jnp.zeros_like(acc_ref)

    acc_ref[0:1, :] += jnp.sum(select, axis=0, keepdims=True)
    acc_ref[1:2, :] += jnp.sum(mask, axis=0, keepdims=True)

    @pl.when(i == NB - 1)
    def _final():
        s = acc_ref[0:1, :] * acc_ref[1:2, :]
        lb = jnp.sum(s) * (float(EXPERT_NUM ** 2) /
                           (float(B) * float(B) * float(EXPERT_NUM)))
        lb_ref[...] = jnp.broadcast_to(lb, (1, 1))


def _router_call(x3, wr_full):
    return pl.pallas_call(
        _router_body,
        grid=(NB,),
        in_specs=[
            pl.BlockSpec((TB, 1, INPUT_DIM), lambda i: (i, 0, 0)),
            pl.BlockSpec((KSZ, EXPERT_NUM), lambda i: (0, 0)),
        ],
        out_specs=[
            pl.BlockSpec((TB, EXPERT_NUM), lambda i: (i, 0)),
            pl.BlockSpec((TB, 1), lambda i: (i, 0)),
            pl.BlockSpec((TB, 1), lambda i: (i, 0)),
            pl.BlockSpec((1, 1), lambda i: (0, 0)),
        ],
        out_shape=[
            jax.ShapeDtypeStruct((B, EXPERT_NUM), jnp.float32),
            jax.ShapeDtypeStruct((B, 1), jnp.float32),
            jax.ShapeDtypeStruct((B, 1), jnp.int32),
            jax.ShapeDtypeStruct((1, 1), jnp.float32),
        ],
        scratch_shapes=[pltpu.VMEM((2, EXPERT_NUM), jnp.float32)],
    )(x3, wr_full)


def _lane_iota():
    return lax.iota(jnp.int32, 16)


def _vgather(v, idx):
    dn = lax.GatherDimensionNumbers(offset_dims=(), collapsed_slice_dims=(0,),
                                    start_index_map=(0,))
    return lax.gather(v, idx[:, None], dn, slice_sizes=(1,),
                      mode=lax.GatherScatterMode.PROMISE_IN_BOUNDS)


def _vsum_splat(v):
    lanes = _lane_iota()
    s = v
    for d in (8, 4, 2, 1):
        s = s + _vgather(s, (lanes + d) & 15)
    return s


def _vcumsum(v):
    lanes = _lane_iota()
    s = v
    for d in (1, 2, 4, 8):
        sh = _vgather(s, jnp.maximum(lanes - d, 0))
        s = s + jnp.where(lanes >= d, sh, 0)
    return s


def _splat_lane(v, lane):
    return _vgather(v, jnp.full((16,), lane, jnp.int32))


def _sc_sort_body(idx_hbm, x_hbm, xs_hbm, dest_hbm, be_hbm,
                  idx_v, dest_v, dest2_v, rows_a, rows_b, be_v,
                  gsem_a, gsem_b, ssem_a, ssem_b):
    nc = jax.lax.axis_size("c")
    wid = lax.axis_index("s") * nc + lax.axis_index("c")
    base = wid * TPW

    pltpu.sync_copy(idx_hbm, idx_v)

    lanes = _lane_iota()
    z16 = jnp.zeros((16,), jnp.int32)

    def hist_step(i, carry):
        tv, pv = carry
        v = idx_v[pl.ds(i * 16, 16)]
        before = jnp.where((i * 16) < base, 1, 0)
        tv2, pv2 = [], []
        for e in range(EXPERT_NUM):
            mi = jnp.where(v == e, 1, 0)
            tv2.append(tv[e] + mi)
            pv2.append(pv[e] + mi * before)
        return tuple(tv2), tuple(pv2)

    tv, pv = lax.fori_loop(0, B // 16, hist_step,
                           ((z16,) * EXPERT_NUM, (z16,) * EXPERT_NUM))
    total = z16
    prior = z16
    for e in range(EXPERT_NUM):
        total = jnp.where(lanes == e, _vsum_splat(tv[e]), total)
        prior = jnp.where(lanes == e, _vsum_splat(pv[e]), prior)

    padded = (total + (GB - 1)) & jnp.int32(~(GB - 1))
    cum = _vcumsum(padded)
    base_e = cum - padded
    pos0 = base_e + prior

    carry = z16
    for c in range(NCH):
        v = idx_v[pl.ds(base + c * 16, 16)]
        dest_c = z16
        for e in range(EXPERT_NUM):
            m = v == e
            mi = jnp.where(m, 1, 0)
            excl = _vcumsum(mi) - mi
            start_e = _splat_lane(pos0 + carry, e)
            dest_c = jnp.where(m, start_e + excl, dest_c)
            cnt = _vsum_splat(mi)
            carry = jnp.where(lanes == e, carry + cnt, carry)
        dest_v[pl.ds(c * 16, 16)] = dest_c
        dest2_v[c, :] = dest_c

    pltpu.sync_copy(dest_v, dest_hbm.at[pl.ds(base, TPW)])

    nsub = TPW // 8
    for j in range(nsub):
        buf = rows_a if j % 2 == 0 else rows_b
        gsem = gsem_a if j % 2 == 0 else gsem_b
        ssem = ssem_a if j % 2 == 0 else ssem_b
        idx8 = dest2_v.at[(j - 2) // 2, pl.ds(((j - 2) % 2) * 8, 8)]
        if j >= 2:
            pltpu.make_async_copy(buf, xs_hbm.at[idx8], ssem).wait()
        pltpu.async_copy(x_hbm.at[pl.ds(base + j * 8, 8)], buf, gsem).wait()
        pltpu.async_copy(
            buf, xs_hbm.at[dest2_v.at[j // 2, pl.ds((j % 2) * 8, 8)]], ssem)
    for j in range(nsub - 2, nsub):
        buf = rows_a if j % 2 == 0 else rows_b
        ssem = ssem_a if j % 2 == 0 else ssem_b
        pltpu.make_async_copy(
            buf, xs_hbm.at[dest2_v.at[j // 2, pl.ds((j % 2) * 8, 8)]],
            ssem).wait()

    @pl.when(wid == 0)
    def _be():
        for c3 in range(BE_PAD // 16):
            posv = (lanes + c3 * 16) * GB
            acc = jnp.zeros((16,), jnp.int32)
            for e in range(EXPERT_NUM):
                ce = _splat_lane(cum, e)
                acc = acc + jnp.where(posv >= ce, 1, 0)
            be_v[pl.ds(c3 * 16, 16)] = jnp.minimum(acc, EXPERT_NUM - 1)
        pltpu.sync_copy(be_v, be_hbm)


def _sc_sort_call(idx_flat, x2):
    mesh = plsc.VectorSubcoreMesh(core_axis_name="c", subcore_axis_name="s")
    f = pl.kernel(
        _sc_sort_body,
        compiler_params=pltpu.CompilerParams(use_tc_tiling_on_sc=True),
        out_type=[
            jax.ShapeDtypeStruct((SORT_CAP, 1, INPUT_DIM), jnp.float32),
            jax.ShapeDtypeStruct((B,), jnp.int32),
            jax.ShapeDtypeStruct((BE_PAD,), jnp.int32),
        ],
        mesh=mesh,
        scratch_types=[
            pltpu.VMEM((B,), jnp.int32),
            pltpu.VMEM((TPW,), jnp.int32),
            pltpu.VMEM((NCH, 16), jnp.int32),
            pltpu.VMEM((8, 1, INPUT_DIM), jnp.float32),
            pltpu.VMEM((8, 1, INPUT_DIM), jnp.float32),
            pltpu.VMEM((BE_PAD,), jnp.int32),
            pltpu.SemaphoreType.DMA,
            pltpu.SemaphoreType.DMA,
            pltpu.SemaphoreType.DMA,
            pltpu.SemaphoreType.DMA,
        ],
    )
    return f(idx_flat, x2)


def _grouped_body(be_sref, x4_ref, wt_ref, b_ref, out_ref):
    xb = x4_ref[...].reshape(GB, INPUT_DIM).astype(jnp.bfloat16)
    wt = wt_ref[0].astype(jnp.bfloat16)
    bias = b_ref[0, 0, :][None, :]
    h3s = jnp.zeros((GB, C2), jnp.float32)
    for p_i in range(PATCH_NUM):
        m = jnp.dot(xb[:, p_i * KSZ:(p_i + 1) * KSZ], wt,
                    preferred_element_type=jnp.float32)
        hh = m + bias
        h3s = h3s + hh * hh * hh
    iota_c = lax.broadcasted_iota(jnp.int32, (C2, 2), 0)
    iota_h = lax.broadcasted_iota(jnp.int32, (C2, 2), 1)
    sel_mat = ((iota_c < OUT_CHANNEL) == (iota_h == 0)).astype(jnp.float32)
    out_ref[...] = jnp.dot(h3s, sel_mat, preferred_element_type=jnp.float32)


def _grouped_call(blk_expert, x4, conv_wt, conv_b):
    grid_spec = pltpu.PrefetchScalarGridSpec(
        num_scalar_prefetch=1,
        grid=(NBLK,),
        in_specs=[
            pl.BlockSpec((GB, 1, INPUT_DIM), lambda i, be: (i, 0, 0)),
            pl.BlockSpec((1, KSZ, C2), lambda i, be: (be[i], 0, 0)),
            pl.BlockSpec((1, 1, C2), lambda i, be: (be[i], 0, 0)),
        ],
        out_specs=pl.BlockSpec((GB, 2), lambda i, be: (i, 0)),
    )
    return pl.pallas_call(
        _grouped_body,
        grid_spec=grid_spec,
        out_shape=jax.ShapeDtypeStruct((SORT_CAP, 2), jnp.float32),
    )(blk_expert, x4, conv_wt, conv_b)


def _combine_body(dest_ref, gate_ref, s_ref, out_ref):
    dest = dest_ref[...]
    iota_s = lax.broadcasted_iota(jnp.int32, (TB, SORT_CAP), 1)
    onehot = (iota_s == dest).astype(jnp.bfloat16)
    sp = jnp.dot(onehot, s_ref[...].astype(jnp.bfloat16),
                 preferred_element_type=jnp.float32)
    logits = gate_ref[...] * sp
    mx = jnp.max(logits, axis=1, keepdims=True)
    p = jnp.exp(logits - mx)
    out_ref[...] = p / jnp.sum(p, axis=1, keepdims=True)


def _combine_call(dest, gate, s_sorted):
    return pl.pallas_call(
        _combine_body,
        grid=(NB,),
        in_specs=[
            pl.BlockSpec((TB, 1), lambda i: (i, 0)),
            pl.BlockSpec((TB, 1), lambda i: (i, 0)),
            pl.BlockSpec((SORT_CAP, 2), lambda i: (0, 0)),
        ],
        out_specs=pl.BlockSpec((TB, 2), lambda i: (i, 0)),
        out_shape=jax.ShapeDtypeStruct((B, 2), jnp.float32),
    )(dest, gate, s_sorted)


@jax.jit
def _moe_pipeline(x, wr_full, conv_wt, conv_b):
    sel0, gate, idx, lb = _router_call(x, wr_full)
    x_sorted, dest, blk_expert = _sc_sort_call(idx.reshape(B), x)
    s_sorted = _grouped_call(blk_expert, x_sorted, conv_wt,
                             conv_b.reshape(EXPERT_NUM, 1, C2))
    out = _combine_call(dest.reshape(B, 1), gate, s_sorted)
    return out, sel0, lb


def kernel(x, router_w, conv_w, conv_b):
    wr_full = router_w[:, 0, :].T
    conv_wt = conv_w[:, :, 0, :].transpose(0, 2, 1)
    out, sel0, lb = _moe_pipeline(x, wr_full, conv_wt, conv_b)
    return out, sel0, lb.reshape(())


# speedup vs baseline: 1.1401x; 1.0245x over previous
import functools

import jax
import jax.numpy as jnp
from jax import lax
from jax.experimental import pallas as pl
from jax.experimental.pallas import tpu as pltpu
from jax.experimental.pallas import tpu_sc as plsc

INPUT_DIM = 4096
PATCH_NUM = 16
KSZ = INPUT_DIM // PATCH_NUM
OUT_CHANNEL = 192
C2 = 2 * OUT_CHANNEL
EXPERT_NUM = 8
B = 2048

TB = 512
NB = B // TB

GB = 64
SORT_CAP = B + EXPERT_NUM * GB
NBLK = SORT_CAP // GB
BE_PAD = 48

NW = 32
TPW = B // NW
NCH = TPW // 16


def _router_body(x_ref, wr_ref, sel0_ref, gate_ref, idx_ref, lb_ref, acc_ref):
    i = pl.program_id(0)
    xb = x_ref[...].reshape(TB, INPUT_DIM)
    ps = xb[:, 0:KSZ]
    for p_i in range(1, PATCH_NUM):
        ps = ps + xb[:, p_i * KSZ:(p_i + 1) * KSZ]
    select = jnp.dot(ps.astype(jnp.bfloat16),
                     wr_ref[...].astype(jnp.bfloat16),
                     preferred_element_type=jnp.float32)

    gate = jnp.max(select, axis=1, keepdims=True)
    iota8 = lax.broadcasted_iota(jnp.int32, (TB, EXPERT_NUM), 1)
    idx = jnp.min(jnp.where(select == gate, iota8, EXPERT_NUM), axis=1,
                  keepdims=True)
    mask = (iota8 == idx).astype(jnp.float32)
    sel0_ref[...] = mask * (gate != 0.0).astype(jnp.float32)
    gate_ref[...] = gate
    idx_ref[...] = idx

    @pl.when(i == 0)
    def _init():
        acc_ref[...] = jnp.zeros_like(acc_ref)

    acc_ref[0:1, :] += jnp.sum(select, axis=0, keepdims=True)
    acc_ref[1:2, :] += jnp.sum(mask, axis=0, keepdims=True)

    @pl.when(i == NB - 1)
    def _final():
        s = acc_ref[0:1, :] * acc_ref[1:2, :]
        lb = jnp.sum(s) * (float(EXPERT_NUM ** 2) /
                           (float(B) * float(B) * float(EXPERT_NUM)))
        lb_ref[...] = jnp.broadcast_to(lb, (1, 1))


def _router_call(x3, wr_full):
    return pl.pallas_call(
        _router_body,
        grid=(NB,),
        in_specs=[
            pl.BlockSpec((TB, 1, INPUT_DIM), lambda i: (i, 0, 0)),
            pl.BlockSpec((KSZ, EXPERT_NUM), lambda i: (0, 0)),
        ],
        out_specs=[
            pl.BlockSpec((TB, EXPERT_NUM), lambda i: (i, 0)),
            pl.BlockSpec((TB, 1), lambda i: (i, 0)),
            pl.BlockSpec((TB, 1), lambda i: (i, 0)),
            pl.BlockSpec((1, 1), lambda i: (0, 0)),
        ],
        out_shape=[
            jax.ShapeDtypeStruct((B, EXPERT_NUM), jnp.float32),
            jax.ShapeDtypeStruct((B, 1), jnp.float32),
            jax.ShapeDtypeStruct((B, 1), jnp.int32),
            jax.ShapeDtypeStruct((1, 1), jnp.float32),
        ],
        scratch_shapes=[pltpu.VMEM((2, EXPERT_NUM), jnp.float32)],
    )(x3, wr_full)


def _lane_iota():
    return lax.iota(jnp.int32, 16)


def _vgather(v, idx):
    dn = lax.GatherDimensionNumbers(offset_dims=(), collapsed_slice_dims=(0,),
                                    start_index_map=(0,))
    return lax.gather(v, idx[:, None], dn, slice_sizes=(1,),
                      mode=lax.GatherScatterMode.PROMISE_IN_BOUNDS)


def _vsum_splat(v):
    lanes = _lane_iota()
    s = v
    for d in (8, 4, 2, 1):
        s = s + _vgather(s, (lanes + d) & 15)
    return s


def _vcumsum(v):
    lanes = _lane_iota()
    s = v
    for d in (1, 2, 4, 8):
        sh = _vgather(s, jnp.maximum(lanes - d, 0))
        s = s + jnp.where(lanes >= d, sh, 0)
    return s


def _splat_lane(v, lane):
    return _vgather(v, jnp.full((16,), lane, jnp.int32))


def _sc_sort_body(idx_hbm, x_hbm, xs_hbm, dest_hbm, be_hbm,
                  idx_v, dest_v, dest2_v, rows_a, rows_b, be_v,
                  gsem_a, gsem_b, ssem_a, ssem_b):
    nc = jax.lax.axis_size("c")
    wid = lax.axis_index("s") * nc + lax.axis_index("c")
    base = wid * TPW

    pltpu.sync_copy(idx_hbm, idx_v)

    lanes = _lane_iota()
    z16 = jnp.zeros((16,), jnp.int32)

    def hist_step(i, carry):
        tv, pv = carry
        v = idx_v[pl.ds(i * 16, 16)]
        before = jnp.where((i * 16) < base, 1, 0)
        tv2, pv2 = [], []
        for e in range(EXPERT_NUM):
            mi = jnp.where(v == e, 1, 0)
            tv2.append(tv[e] + mi)
            pv2.append(pv[e] + mi * before)
        return tuple(tv2), tuple(pv2)

    tv, pv = lax.fori_loop(0, B // 16, hist_step,
                           ((z16,) * EXPERT_NUM, (z16,) * EXPERT_NUM))
    total = z16
    prior = z16
    for e in range(EXPERT_NUM):
        total = jnp.where(lanes == e, _vsum_splat(tv[e]), total)
        prior = jnp.where(lanes == e, _vsum_splat(pv[e]), prior)

    padded = (total + (GB - 1)) & jnp.int32(~(GB - 1))
    cum = _vcumsum(padded)
    base_e = cum - padded
    pos0 = base_e + prior

    carry = z16
    for c in range(NCH):
        v = idx_v[pl.ds(base + c * 16, 16)]
        dest_c = z16
        for e in range(EXPERT_NUM):
            m = v == e
            mi = jnp.where(m, 1, 0)
            excl = _vcumsum(mi) - mi
            start_e = _splat_lane(pos0 + carry, e)
            dest_c = jnp.where(m, start_e + excl, dest_c)
            cnt = _vsum_splat(mi)
            carry = jnp.where(lanes == e, carry + cnt, carry)
        dest_v[pl.ds(c * 16, 16)] = dest_c
        dest2_v[c, :] = dest_c

    pltpu.sync_copy(dest_v, dest_hbm.at[pl.ds(base, TPW)])

    nsub = TPW // 8
    for j in range(nsub):
        buf = rows_a if j % 2 == 0 else rows_b
        gsem = gsem_a if j % 2 == 0 else gsem_b
        ssem = ssem_a if j % 2 == 0 else ssem_b
        idx8 = dest2_v.at[(j - 2) // 2, pl.ds(((j - 2) % 2) * 8, 8)]
        if j >= 2:
            pltpu.make_async_copy(buf, xs_hbm.at[idx8], ssem).wait()
        pltpu.async_copy(x_hbm.at[pl.ds(base + j * 8, 8)], buf, gsem).wait()
        pltpu.async_copy(
            buf, xs_hbm.at[dest2_v.at[j // 2, pl.ds((j % 2) * 8, 8)]], ssem)
    for j in range(nsub - 2, nsub):
        buf = rows_a if j % 2 == 0 else rows_b
        ssem = ssem_a if j % 2 == 0 else ssem_b
        pltpu.make_async_copy(
            buf, xs_hbm.at[dest2_v.at[j // 2, pl.ds((j % 2) * 8, 8)]],
            ssem).wait()

    @pl.when(wid == 0)
    def _be():
        for c3 in range(BE_PAD // 16):
            posv = (lanes + c3 * 16) * GB
            acc = jnp.zeros((16,), jnp.int32)
            for e in range(EXPERT_NUM):
                ce = _splat_lane(cum, e)
                acc = acc + jnp.where(posv >= ce, 1, 0)
            be_v[pl.ds(c3 * 16, 16)] = jnp.minimum(acc, EXPERT_NUM - 1)
        pltpu.sync_copy(be_v, be_hbm)


def _sc_sort_call(idx_flat, x2):
    mesh = plsc.VectorSubcoreMesh(core_axis_name="c", subcore_axis_name="s")
    f = pl.kernel(
        _sc_sort_body,
        compiler_params=pltpu.CompilerParams(use_tc_tiling_on_sc=True),
        out_type=[
            jax.ShapeDtypeStruct((SORT_CAP, 1, INPUT_DIM), jnp.float32),
            jax.ShapeDtypeStruct((B,), jnp.int32),
            jax.ShapeDtypeStruct((BE_PAD,), jnp.int32),
        ],
        mesh=mesh,
        scratch_types=[
            pltpu.VMEM((B,), jnp.int32),
            pltpu.VMEM((TPW,), jnp.int32),
            pltpu.VMEM((NCH, 16), jnp.int32),
            pltpu.VMEM((8, 1, INPUT_DIM), jnp.float32),
            pltpu.VMEM((8, 1, INPUT_DIM), jnp.float32),
            pltpu.VMEM((BE_PAD,), jnp.int32),
            pltpu.SemaphoreType.DMA,
            pltpu.SemaphoreType.DMA,
            pltpu.SemaphoreType.DMA,
            pltpu.SemaphoreType.DMA,
        ],
    )
    return f(idx_flat, x2)


def _grouped_body(be_sref, x4_ref, wt_ref, b_ref, out_ref):
    xb = x4_ref[...].reshape(GB, INPUT_DIM).astype(jnp.bfloat16)
    wt = wt_ref[0].astype(jnp.bfloat16)
    bias = b_ref[0, 0, :][None, :]
    h3s = jnp.zeros((GB, C2), jnp.float32)
    for p_i in range(PATCH_NUM):
        m = jnp.dot(xb[:, p_i * KSZ:(p_i + 1) * KSZ], wt,
                    preferred_element_type=jnp.float32)
        hh = m + bias
        h3s = h3s + hh * hh * hh
    iota_c = lax.broadcasted_iota(jnp.int32, (C2, 2), 0)
    iota_h = lax.broadcasted_iota(jnp.int32, (C2, 2), 1)
    sel_mat = ((iota_c < OUT_CHANNEL) == (iota_h == 0)).astype(jnp.float32)
    out_ref[...] = jnp.dot(h3s, sel_mat, preferred_element_type=jnp.float32)


def _grouped_call(blk_expert, x4, conv_wt, conv_b):
    grid_spec = pltpu.PrefetchScalarGridSpec(
        num_scalar_prefetch=1,
        grid=(NBLK,),
        in_specs=[
            pl.BlockSpec((GB, 1, INPUT_DIM), lambda i, be: (i, 0, 0)),
            pl.BlockSpec((1, KSZ, C2), lambda i, be: (be[i], 0, 0)),
            pl.BlockSpec((1, 1, C2), lambda i, be: (be[i], 0, 0)),
        ],
        out_specs=pl.BlockSpec((GB, 2), lambda i, be: (i, 0)),
    )
    return pl.pallas_call(
        _grouped_body,
        grid_spec=grid_spec,
        out_shape=jax.ShapeDtypeStruct((SORT_CAP, 2), jnp.float32),
    )(blk_expert, x4, conv_wt, conv_b)


def _combine_body(dest_ref, gate_ref, s_ref, out_ref):
    dest = dest_ref[...]
    iota_s = lax.broadcasted_iota(jnp.int32, (TB, SORT_CAP), 1)
    onehot = (iota_s == dest).astype(jnp.bfloat16)
    sp = jnp.dot(onehot, s_ref[...].astype(jnp.bfloat16),
                 preferred_element_type=jnp.float32)
    logits = gate_ref[...] * sp
    mx = jnp.max(logits, axis=1, keepdims=True)
    p = jnp.exp(logits - mx)
    out_ref[...] = p / jnp.sum(p, axis=1, keepdims=True)


def _combine_call(dest, gate, s_sorted):
    return pl.pallas_call(
        _combine_body,
        grid=(NB,),
        in_specs=[
            pl.BlockSpec((TB, 1), lambda i: (i, 0)),
            pl.BlockSpec((TB, 1), lambda i: (i, 0)),
            pl.BlockSpec((SORT_CAP, 2), lambda i: (0, 0)),
        ],
        out_specs=pl.BlockSpec((TB, 2), lambda i: (i, 0)),
        out_shape=jax.ShapeDtypeStruct((B, 2), jnp.float32),
    )(dest, gate, s_sorted)


@jax.jit
def _moe_pipeline(x, wr_full, conv_wt, conv_b):
    sel0, gate, idx, lb = _router_call(x, wr_full)
    x_sorted, dest, blk_expert = _sc_sort_call(idx.reshape(B), x)
    s_sorted = _grouped_call(blk_expert, x_sorted, conv_wt,
                             conv_b.reshape(EXPERT_NUM, 1, C2))
    out = _combine_call(dest.reshape(B, 1), gate, s_sorted)
    return out, sel0, lb


def kernel(x, router_w, conv_w, conv_b):
    wr_full = router_w[:, 0, :].T
    conv_wt = conv_w[:, :, 0, :].transpose(0, 2, 1)
    out, sel0, lb = _moe_pipeline(x, wr_full, conv_wt, conv_b)
    return out, sel0, lb.reshape(())
